# baseline (speedup 1.0000x reference)
import functools

import jax
import jax.numpy as jnp
from jax import lax
from jax.experimental import pallas as pl
from jax.experimental.pallas import tpu as pltpu
from jax.experimental.pallas import tpu_sc as plsc

T = 8
B = 10000
IN_DIM = 128
H1 = 32
HEADS = 4
GRU = 64
PW = 5
E = 160000

NC = 2
NS = 16
NW = NC * NS
CH = 96
SUP = 960
CPS = SUP // CH
EPW = 10560
EPAD = NS * EPW
NSUP = EPW // SUP
HB = 5120
BPAD = NC * HB
RPT = HB // NS
FD = HEADS * H1
SENT = -1


def _sc_edge_body(h_hbm, s1_hbm, s2_hbm, src_hbm, dst_hbm, out_hbm, z_hbm,
                  s1_v, s2_v, isrc, idst, iloc,
                  rows0, rows1, zrow, gsem0, gsem1, acc):
    c = lax.axis_index("c")
    s = lax.axis_index("s")
    wid = s * NC + c

    pltpu.sync_copy(s1_hbm, s1_v)
    pltpu.sync_copy(s2_hbm.at[pl.ds(c * (HB * HEADS), HB * HEADS)], s2_v)

    def _zero_row(i, carry):
        for g in range(FD // 16):
            rows0[i, pl.ds(g * 16, 16)] = jnp.zeros((16,), jnp.float32)
        return carry

    lax.fori_loop(0, CH, _zero_row, 0)
    base = s * RPT
    for k in range(RPT // CH):
        pltpu.sync_copy(rows0, acc.at[pl.ds(base + k * CH, CH)])
    tail = RPT % CH
    if tail:
        pltpu.sync_copy(rows0.at[pl.ds(0, tail)],
                        acc.at[pl.ds(base + (RPT // CH) * CH, tail)])
    plsc.subcore_barrier()

    ebase = s * EPW
    rlo = c * HB
    zzero = jnp.zeros((16,), jnp.float32)

    def _compute_chunk(goff, loff, rows, iloc, zcarry):
        def _group(g, zc):
            o16 = pl.multiple_of(loff + g * 16, 16)
            src16 = isrc[pl.ds(o16, 16)]
            dst16 = idst[pl.ds(o16, 16)]
            gidx = lax.broadcasted_iota(jnp.int32, (16,), 0) + (goff + g * 16)
            ldst = dst16 - rlo
            owned = (ldst >= 0) & (ldst < HB)
            inr = (gidx < E) & owned
            iloc[pl.ds(pl.multiple_of(g * 16, 16), 16)] = jnp.where(inr, ldst, SENT)
            newz = []
            ws = []
            src4 = src16 * HEADS
            dst4 = jnp.where(owned, ldst, 0) * HEADS
            for h in range(HEADS):
                sa = plsc.load_gather(s1_v, [src4 + h])
                sb = plsc.load_gather(s2_v, [dst4 + h])
                u = sa + sb
                u = jnp.where(u >= 0.0, u, 0.01 * u)
                w = jnp.exp(u)
                ws.append(w)
                newz.append(zc[h] + jnp.where(inr, w, 0.0))
            for j in range(16):
                r = g * 16 + j
                for gg in range(FD // 16):
                    wsc = ws[gg // 2][j]
                    rows[r, pl.ds(gg * 16, 16)] = rows[r, pl.ds(gg * 16, 16)] * wsc
            return tuple(newz)

        return lax.fori_loop(0, CH // 16, _group, zcarry)

    def _super(i, zcarry):
        soff = ebase + i * SUP
        pltpu.sync_copy(src_hbm.at[pl.ds(soff, SUP)], isrc)
        pltpu.sync_copy(dst_hbm.at[pl.ds(soff, SUP)], idst)
        pltpu.async_copy(h_hbm.at[isrc.at[pl.ds(0, CH)]], rows0, gsem0)

        def _pos(k, pos, rows, sem, rows_n, sem_n, zc):
            cidx = 2 * k + pos
            loff = cidx * CH
            pltpu.make_async_copy(h_hbm.at[pl.ds(0, CH)], rows, sem).wait()

            def _issue():
                noff = pl.multiple_of((cidx + 1) * CH, CH)
                pltpu.async_copy(h_hbm.at[isrc.at[pl.ds(noff, CH)]],
                                 rows_n, sem_n)

            if pos == 0:
                _issue()
            else:
                @pl.when(k < CPS // 2 - 1)
                def _():
                    _issue()

            zc = _compute_chunk(soff + loff, loff, rows, iloc, zc)
            pltpu.sync_copy(
                rows, acc.at[plsc.Indices(iloc, ignored_value=SENT)], add=True)
            return zc

        def _pair(k, zc):
            zc = _pos(k, 0, rows0, gsem0, rows1, gsem1, zc)
            zc = _pos(k, 1, rows1, gsem1, rows0, gsem0, zc)
            return zc

        return lax.fori_loop(0, CPS // 2, _pair, zcarry)

    zfin = lax.fori_loop(0, NSUP, _super, (zzero,) * HEADS)

    for g in range(8):
        ztot = jnp.sum(zfin[g // 2])
        zrow[pl.ds(g * 16, 16)] = jnp.full((16,), 1.0, jnp.float32) * ztot
    pltpu.sync_copy(zrow, z_hbm.at[wid])

    plsc.subcore_barrier()
    pltpu.sync_copy(acc.at[pl.ds(base, RPT)], out_hbm.at[c, pl.ds(base, RPT)])


_sc_edge = functools.partial(
    pl.kernel,
    out_type=(jax.ShapeDtypeStruct((NC, HB, FD), jnp.float32),
              jax.ShapeDtypeStruct((NW, FD), jnp.float32)),
    mesh=plsc.VectorSubcoreMesh(core_axis_name="c", subcore_axis_name="s"),
    compiler_params=pltpu.CompilerParams(needs_layout_passes=False),
    scratch_types=[
        pltpu.VMEM((B * HEADS,), jnp.float32),
        pltpu.VMEM((HB * HEADS,), jnp.float32),
        pltpu.VMEM((SUP,), jnp.int32),
        pltpu.VMEM((SUP,), jnp.int32),
        pltpu.VMEM((CH,), jnp.int32),
        pltpu.VMEM((CH, FD), jnp.float32),
        pltpu.VMEM((CH, FD), jnp.float32),
        pltpu.VMEM((FD,), jnp.float32),
        pltpu.SemaphoreType.DMA,
        pltpu.SemaphoreType.DMA,
        pltpu.VMEM_SHARED((HB, FD), jnp.float32),
    ],
)(_sc_edge_body)


def _tc1_body(x_ref, w_ref, a_ref, h_ref, s_ref):
    h = jnp.dot(x_ref[...], w_ref[...], preferred_element_type=jnp.float32)
    h_ref[...] = h
    s_ref[...] = jnp.dot(h, a_ref[...], preferred_element_type=jnp.float32)


def _tc1(x, wt, amat, bm):
    m = x.shape[0]
    grid = m // bm
    return pl.pallas_call(
        _tc1_body,
        grid=(grid,),
        in_specs=[
            pl.BlockSpec((bm, IN_DIM), lambda i: (i, 0)),
            pl.BlockSpec((IN_DIM, FD), lambda i: (0, 0)),
            pl.BlockSpec((FD, 2 * HEADS), lambda i: (0, 0)),
        ],
        out_specs=[
            pl.BlockSpec((bm, FD), lambda i: (i, 0)),
            pl.BlockSpec((bm, 2 * HEADS), lambda i: (i, 0)),
        ],
        out_shape=[
            jax.ShapeDtypeStruct((m, FD), jnp.float32),
            jax.ShapeDtypeStruct((m, 2 * HEADS), jnp.float32),
        ],
    )(x, wt, amat)


def _elu(x):
    return jnp.where(x > 0.0, x, jnp.exp(x) - 1.0)


def _tc2_body(o_ref, z_ref, w_ref, a_ref, h_ref, s_ref):
    zsum = jnp.dot(jnp.ones((1, NW), jnp.float32), z_ref[...],
                   preferred_element_type=jnp.float32)
    x = _elu(o_ref[...] / zsum)
    h = jnp.dot(x, w_ref[...], preferred_element_type=jnp.float32)
    h_ref[...] = h
    s_ref[...] = jnp.dot(h, a_ref[...], preferred_element_type=jnp.float32)


def _tc2(oflat, z, wt, amat, bm):
    grid = B // bm
    return pl.pallas_call(
        _tc2_body,
        grid=(grid,),
        in_specs=[
            pl.BlockSpec((bm, FD), lambda i: (i, 0)),
            pl.BlockSpec((NW, FD), lambda i: (0, 0)),
            pl.BlockSpec((FD, FD), lambda i: (0, 0)),
            pl.BlockSpec((FD, 2 * HEADS), lambda i: (0, 0)),
        ],
        out_specs=[
            pl.BlockSpec((bm, FD), lambda i: (i, 0)),
            pl.BlockSpec((bm, 2 * HEADS), lambda i: (i, 0)),
        ],
        out_shape=[
            jax.ShapeDtypeStruct((B, FD), jnp.float32),
            jax.ShapeDtypeStruct((B, 2 * HEADS), jnp.float32),
        ],
    )(oflat, z, wt, amat)


def _tc3_body(o_ref, z_ref, hp_ref, ci_ref, cr_ref, it_ref, rt_ref,
              n_ref, wir_ref, wiz_ref, win_ref, whr_ref, whz_ref, whn_ref,
              brz_ref, bn_ref, wio_ref, cio_ref, bio_ref, wro_ref, cro_ref,
              bro_ref, wso_ref, cso_ref, bso_ref,
              hn_ref, pi_ref, pr_ref, fi_ref, fr_ref):
    zsum = jnp.dot(jnp.ones((1, NW), jnp.float32), z_ref[...],
                   preferred_element_type=jnp.float32)
    x = _elu(o_ref[...] / zsum)
    hp = hp_ref[...]
    brz = brz_ref[...]
    r = jax.nn.sigmoid(
        jnp.dot(x, wir_ref[...], preferred_element_type=jnp.float32)
        + jnp.dot(hp, whr_ref[...], preferred_element_type=jnp.float32)
        + brz[0:1, :])
    zg = jax.nn.sigmoid(
        jnp.dot(x, wiz_ref[...], preferred_element_type=jnp.float32)
        + jnp.dot(hp, whz_ref[...], preferred_element_type=jnp.float32)
        + brz[1:2, :])
    bn = bn_ref[...]
    hcand = jnp.dot(hp, whn_ref[...], preferred_element_type=jnp.float32) + bn[1:2, :]
    n = jnp.tanh(
        jnp.dot(x, win_ref[...], preferred_element_type=jnp.float32)
        + bn[0:1, :] + r * hcand)
    hn = (1.0 - zg) * n + zg * hp
    hn_ref[...] = hn

    ci = ci_ref[...]
    cr = cr_ref[...]
    cio = cio_ref[...]
    pi_ref[...] = (jnp.dot(hn, wio_ref[...], preferred_element_type=jnp.float32)
                   + ci * cio[0:1, :] + cr * cio[1:2, :] + bio_ref[...])
    cro = cro_ref[...]
    pr_ref[...] = (jnp.dot(hn, wro_ref[...], preferred_element_type=jnp.float32)
                   + ci * cro[0:1, :] + cr * cro[1:2, :] + bro_ref[...])
    cso = cso_ref[...]
    sir = jax.nn.sigmoid(
        jnp.dot(hn, wso_ref[...], preferred_element_type=jnp.float32)
        + ci * cso[0:1, :] + cr * cso[1:2, :] + bso_ref[...])
    al = sir[:, 0:1]
    be = sir[:, 1:2]
    it = it_ref[...]
    nn = n_ref[...]
    ssus = jnp.clip(nn - it - rt_ref[...], 0.0, None)
    fi_ref[...] = al * it * (ssus / nn) - be * it
    fr_ref[...] = be * it


def _tc3(oflat, z, hp, ci, cr, it, rt, nn, wd, bm):
    grid = B // bm

    def rep(shape):
        return pl.BlockSpec(shape, lambda i: (0, 0))

    def blk(width):
        return pl.BlockSpec((bm, width), lambda i: (i, 0))

    return pl.pallas_call(
        _tc3_body,
        grid=(grid,),
        in_specs=[
            blk(FD),
            rep((NW, FD)), blk(GRU),
            blk(1), blk(1), blk(1), blk(1), blk(1),
            rep((FD, GRU)), rep((FD, GRU)), rep((FD, GRU)),
            rep((GRU, GRU)), rep((GRU, GRU)), rep((GRU, GRU)),
            rep((2, GRU)), rep((2, GRU)),
            rep((GRU, PW)), rep((2, PW)), rep((1, PW)),
            rep((GRU, PW)), rep((2, PW)), rep((1, PW)),
            rep((GRU, 2)), rep((2, 2)), rep((1, 2)),
        ],
        out_specs=[
            blk(GRU), blk(PW), blk(PW), blk(1), blk(1),
        ],
        out_shape=[
            jax.ShapeDtypeStruct((B, GRU), jnp.float32),
            jax.ShapeDtypeStruct((B, PW), jnp.float32),
            jax.ShapeDtypeStruct((B, PW), jnp.float32),
            jax.ShapeDtypeStruct((B, 1), jnp.float32),
            jax.ShapeDtypeStruct((B, 1), jnp.float32),
        ],
    )(oflat, z, hp, ci, cr, it, rt, nn, *wd)


def _attn_mat(a):
    amat = jnp.zeros((FD, 2 * HEADS), jnp.float32)
    for h in range(HEADS):
        amat = amat.at[H1 * h:H1 * (h + 1), h].set(a[h, :H1])
        amat = amat.at[H1 * h:H1 * (h + 1), h + HEADS].set(a[h, H1:])
    return amat


def kernel(dynamic, cI, cR, N, I, R, edge_index, W1, a1, W2, a2, Wih, Whh,
           bih, bhh, WI, bI, WR, bR, Wsir, bsir):
    f32 = jnp.float32
    src = edge_index[0]
    dst = edge_index[1]
    pad = jnp.zeros((EPAD - E,), jnp.int32)
    srcp = jnp.concatenate([src, pad])
    dstp = jnp.concatenate([dst, pad])

    w1t = W1.reshape(FD, IN_DIM).T
    w2t = W2.reshape(FD, FD).T
    a1m = _attn_mat(a1)
    a2m = _attn_mat(a2)

    wir = Wih[0:GRU].T
    wiz = Wih[GRU:2 * GRU].T
    win = Wih[2 * GRU:].T
    whr = Whh[0:GRU].T
    whz = Whh[GRU:2 * GRU].T
    whn = Whh[2 * GRU:].T
    brz = jnp.stack([bih[0:GRU] + bhh[0:GRU], bih[GRU:2 * GRU] + bhh[GRU:2 * GRU]])
    bn = jnp.stack([bih[2 * GRU:], bhh[2 * GRU:]])
    wd = (wir, wiz, win, whr, whz, whn, brz, bn,
          WI[:, :GRU].T, WI[:, GRU:].T, bI.reshape(1, PW),
          WR[:, :GRU].T, WR[:, GRU:].T, bR.reshape(1, PW),
          Wsir[:, :GRU].T, Wsir[:, GRU:].T, bsir.reshape(1, 2))

    def split_scores(s):
        s1f = s[:, :HEADS].reshape(B * HEADS)
        s2f = jnp.concatenate(
            [s[:, HEADS:], jnp.zeros((BPAD - B, HEADS), f32)]).reshape(-1)
        return s1f, s2f

    h1_all, sc1_all = _tc1(dynamic.reshape(T * B, IN_DIM), w1t, a1m, 2000)
    h1_all = h1_all.reshape(T, B, FD)
    sc1_all = sc1_all.reshape(T, B, 2 * HEADS)

    h = jnp.zeros((B, GRU), f32)
    pIs, pRs, fIs, fRs = [], [], [], []
    for t in range(T):
        s1f, s2f = split_scores(sc1_all[t])
        o1, z1 = _sc_edge(h1_all[t], s1f, s2f, srcp, dstp)
        h2, sc2 = _tc2(o1.reshape(BPAD, FD), z1, w2t, a2m, 2000)
        s1f, s2f = split_scores(sc2)
        o2, z2 = _sc_edge(h2, s1f, s2f, srcp, dstp)
        h, pI, pR, fI, fR = _tc3(o2.reshape(BPAD, FD), z2, h, cI[t], cR[t],
                                 I[t], R[t], N, wd, 2000)
        pIs.append(pI)
        pRs.append(pR)
        fIs.append(fI)
        fRs.append(fR)

    pred_I = jnp.stack(pIs, axis=0)
    pred_R = jnp.stack(pRs, axis=0)
    phy_I = jnp.broadcast_to(jnp.stack(fIs, axis=0), pred_I.shape)
    phy_R = jnp.broadcast_to(jnp.stack(fRs, axis=0), pred_R.shape)
    return (pred_I, pred_R, phy_I, phy_R, h)


# speedup vs baseline: 1.7208x; 1.7208x over previous
import functools

import jax
import jax.numpy as jnp
from jax import lax
from jax.experimental import pallas as pl
from jax.experimental.pallas import tpu as pltpu
from jax.experimental.pallas import tpu_sc as plsc

T = 8
B = 10000
IN_DIM = 128
H1 = 32
HEADS = 4
GRU = 64
PW = 5
E = 160000

NC = 2
NS = 16
NW = NC * NS
CH = 64
SUP = 1024
CPS = SUP // CH
EPW = 10240
EPAD = NS * EPW
NSUP = EPW // SUP
HB = 5120
BPAD = NC * HB
RPT = HB // NS
FD = HEADS * H1
SENT = -1


def _sc_edge_body(h_hbm, s1_hbm, s2_hbm, src_hbm, dst_hbm, out_hbm, z_hbm,
                  s1_v, s2_v, isrc, idst, iloc,
                  rows0, rows1, zrow, gsem0, gsem1, acc):
    c = lax.axis_index("c")
    s = lax.axis_index("s")
    wid = s * NC + c

    pltpu.sync_copy(s1_hbm, s1_v)
    pltpu.sync_copy(s2_hbm.at[pl.ds(c * (HB * HEADS), HB * HEADS)], s2_v)

    def _zero_row(i, carry):
        for g in range(FD // 16):
            rows0[i, pl.ds(g * 16, 16)] = jnp.zeros((16,), jnp.float32)
        return carry

    lax.fori_loop(0, CH, _zero_row, 0)
    base = s * RPT
    for k in range(RPT // CH):
        pltpu.sync_copy(rows0, acc.at[pl.ds(base + k * CH, CH)])
    tail = RPT % CH
    if tail:
        pltpu.sync_copy(rows0.at[pl.ds(0, tail)],
                        acc.at[pl.ds(base + (RPT // CH) * CH, tail)])
    plsc.subcore_barrier()

    ebase = s * EPW
    rlo = c * HB
    zzero = jnp.zeros((16,), jnp.float32)

    def _compute_chunk(goff, loff, rows, iloc, zcarry):
        def _group(g, zc):
            o16 = pl.multiple_of(loff + g * 16, 16)
            src16 = isrc[pl.ds(o16, 16)]
            dst16 = idst[pl.ds(o16, 16)]
            gidx = lax.broadcasted_iota(jnp.int32, (16,), 0) + (goff + g * 16)
            ldst = dst16 - rlo
            owned = (ldst >= 0) & (ldst < HB)
            inr = (gidx < E) & owned
            iloc[pl.ds(pl.multiple_of(g * 16, 16), 16)] = jnp.where(inr, ldst, SENT)
            newz = []
            ws = []
            src4 = src16 * HEADS
            dst4 = jnp.where(owned, ldst, 0) * HEADS
            for h in range(HEADS):
                sa = plsc.load_gather(s1_v, [src4 + h])
                sb = plsc.load_gather(s2_v, [dst4 + h])
                u = sa + sb
                u = jnp.where(u >= 0.0, u, 0.01 * u)
                w = jnp.exp(u)
                ws.append(w)
                newz.append(zc[h] + jnp.where(inr, w, 0.0))
            for j in range(16):
                r = g * 16 + j
                for gg in range(FD // 16):
                    wsc = ws[gg // 2][j]
                    rows[r, pl.ds(gg * 16, 16)] = rows[r, pl.ds(gg * 16, 16)] * wsc
            return tuple(newz)

        return lax.fori_loop(0, CH // 16, _group, zcarry)

    def _super(i, zcarry):
        soff = ebase + i * SUP
        pltpu.sync_copy(src_hbm.at[pl.ds(soff, SUP)], isrc)
        pltpu.sync_copy(dst_hbm.at[pl.ds(soff, SUP)], idst)
        pltpu.async_copy(h_hbm.at[isrc.at[pl.ds(0, CH)]], rows0, gsem0)

        def _pos(k, pos, rows, sem, rows_n, sem_n, zc):
            cidx = 2 * k + pos
            loff = cidx * CH
            pltpu.make_async_copy(h_hbm.at[pl.ds(0, CH)], rows, sem).wait()

            def _issue():
                noff = pl.multiple_of((cidx + 1) * CH, CH)
                pltpu.async_copy(h_hbm.at[isrc.at[pl.ds(noff, CH)]],
                                 rows_n, sem_n)

            if pos == 0:
                _issue()
            else:
                @pl.when(k < CPS // 2 - 1)
                def _():
                    _issue()

            zc = _compute_chunk(soff + loff, loff, rows, iloc, zc)
            pltpu.sync_copy(
                rows, acc.at[plsc.Indices(iloc, ignored_value=SENT)], add=True)
            return zc

        def _pair(k, zc):
            zc = _pos(k, 0, rows0, gsem0, rows1, gsem1, zc)
            zc = _pos(k, 1, rows1, gsem1, rows0, gsem0, zc)
            return zc

        return lax.fori_loop(0, CPS // 2, _pair, zcarry)

    zfin = lax.fori_loop(0, NSUP, _super, (zzero,) * HEADS)

    for g in range(8):
        ztot = jnp.sum(zfin[g // 2])
        zrow[pl.ds(g * 16, 16)] = jnp.full((16,), 1.0, jnp.float32) * ztot
    pltpu.sync_copy(zrow, z_hbm.at[wid])

    plsc.subcore_barrier()
    pltpu.sync_copy(acc.at[pl.ds(base, RPT)], out_hbm.at[c, pl.ds(base, RPT)])


_sc_edge = functools.partial(
    pl.kernel,
    out_type=(jax.ShapeDtypeStruct((NC, HB, FD), jnp.float32),
              jax.ShapeDtypeStruct((NW, FD), jnp.float32)),
    mesh=plsc.VectorSubcoreMesh(core_axis_name="c", subcore_axis_name="s"),
    compiler_params=pltpu.CompilerParams(needs_layout_passes=False),
    scratch_types=[
        pltpu.VMEM((B * HEADS,), jnp.float32),
        pltpu.VMEM((HB * HEADS,), jnp.float32),
        pltpu.VMEM((SUP,), jnp.int32),
        pltpu.VMEM((SUP,), jnp.int32),
        pltpu.VMEM((CH,), jnp.int32),
        pltpu.VMEM((CH, FD), jnp.float32),
        pltpu.VMEM((CH, FD), jnp.float32),
        pltpu.VMEM((FD,), jnp.float32),
        pltpu.SemaphoreType.DMA,
        pltpu.SemaphoreType.DMA,
        pltpu.VMEM_SHARED((HB, FD), jnp.float32),
    ],
)(_sc_edge_body)


def _tc1_body(x_ref, w_ref, a_ref, h_ref, s_ref):
    h = jnp.dot(x_ref[...], w_ref[...], preferred_element_type=jnp.float32)
    h_ref[...] = h
    s_ref[...] = jnp.dot(h, a_ref[...], preferred_element_type=jnp.float32)


def _tc1(x, wt, amat, bm):
    m = x.shape[0]
    grid = m // bm
    return pl.pallas_call(
        _tc1_body,
        grid=(grid,),
        in_specs=[
            pl.BlockSpec((bm, IN_DIM), lambda i: (i, 0)),
            pl.BlockSpec((IN_DIM, FD), lambda i: (0, 0)),
            pl.BlockSpec((FD, 2 * HEADS), lambda i: (0, 0)),
        ],
        out_specs=[
            pl.BlockSpec((bm, FD), lambda i: (i, 0)),
            pl.BlockSpec((bm, 2 * HEADS), lambda i: (i, 0)),
        ],
        out_shape=[
            jax.ShapeDtypeStruct((m, FD), jnp.float32),
            jax.ShapeDtypeStruct((m, 2 * HEADS), jnp.float32),
        ],
    )(x, wt, amat)


def _elu(x):
    return jnp.where(x > 0.0, x, jnp.exp(x) - 1.0)


def _tc2_body(o_ref, z_ref, w_ref, a_ref, h_ref, s_ref):
    zsum = jnp.dot(jnp.ones((1, NW), jnp.float32), z_ref[...],
                   preferred_element_type=jnp.float32)
    x = _elu(o_ref[...] / zsum)
    h = jnp.dot(x, w_ref[...], preferred_element_type=jnp.float32)
    h_ref[...] = h
    s_ref[...] = jnp.dot(h, a_ref[...], preferred_element_type=jnp.float32)


def _tc2(oflat, z, wt, amat, bm):
    grid = B // bm
    return pl.pallas_call(
        _tc2_body,
        grid=(grid,),
        in_specs=[
            pl.BlockSpec((bm, FD), lambda i: (i, 0)),
            pl.BlockSpec((NW, FD), lambda i: (0, 0)),
            pl.BlockSpec((FD, FD), lambda i: (0, 0)),
            pl.BlockSpec((FD, 2 * HEADS), lambda i: (0, 0)),
        ],
        out_specs=[
            pl.BlockSpec((bm, FD), lambda i: (i, 0)),
            pl.BlockSpec((bm, 2 * HEADS), lambda i: (i, 0)),
        ],
        out_shape=[
            jax.ShapeDtypeStruct((B, FD), jnp.float32),
            jax.ShapeDtypeStruct((B, 2 * HEADS), jnp.float32),
        ],
    )(oflat, z, wt, amat)


def _tc3_body(o_ref, z_ref, hp_ref, ci_ref, cr_ref, it_ref, rt_ref,
              n_ref, wir_ref, wiz_ref, win_ref, whr_ref, whz_ref, whn_ref,
              brz_ref, bn_ref, wio_ref, cio_ref, bio_ref, wro_ref, cro_ref,
              bro_ref, wso_ref, cso_ref, bso_ref,
              hn_ref, pi_ref, pr_ref, fi_ref, fr_ref):
    zsum = jnp.dot(jnp.ones((1, NW), jnp.float32), z_ref[...],
                   preferred_element_type=jnp.float32)
    x = _elu(o_ref[...] / zsum)
    hp = hp_ref[...]
    brz = brz_ref[...]
    r = jax.nn.sigmoid(
        jnp.dot(x, wir_ref[...], preferred_element_type=jnp.float32)
        + jnp.dot(hp, whr_ref[...], preferred_element_type=jnp.float32)
        + brz[0:1, :])
    zg = jax.nn.sigmoid(
        jnp.dot(x, wiz_ref[...], preferred_element_type=jnp.float32)
        + jnp.dot(hp, whz_ref[...], preferred_element_type=jnp.float32)
        + brz[1:2, :])
    bn = bn_ref[...]
    hcand = jnp.dot(hp, whn_ref[...], preferred_element_type=jnp.float32) + bn[1:2, :]
    n = jnp.tanh(
        jnp.dot(x, win_ref[...], preferred_element_type=jnp.float32)
        + bn[0:1, :] + r * hcand)
    hn = (1.0 - zg) * n + zg * hp
    hn_ref[...] = hn

    ci = ci_ref[...]
    cr = cr_ref[...]
    cio = cio_ref[...]
    pi_ref[...] = (jnp.dot(hn, wio_ref[...], preferred_element_type=jnp.float32)
                   + ci * cio[0:1, :] + cr * cio[1:2, :] + bio_ref[...])
    cro = cro_ref[...]
    pr_ref[...] = (jnp.dot(hn, wro_ref[...], preferred_element_type=jnp.float32)
                   + ci * cro[0:1, :] + cr * cro[1:2, :] + bro_ref[...])
    cso = cso_ref[...]
    sir = jax.nn.sigmoid(
        jnp.dot(hn, wso_ref[...], preferred_element_type=jnp.float32)
        + ci * cso[0:1, :] + cr * cso[1:2, :] + bso_ref[...])
    al = sir[:, 0:1]
    be = sir[:, 1:2]
    it = it_ref[...]
    nn = n_ref[...]
    ssus = jnp.clip(nn - it - rt_ref[...], 0.0, None)
    fi_ref[...] = al * it * (ssus / nn) - be * it
    fr_ref[...] = be * it


def _tc3(oflat, z, hp, ci, cr, it, rt, nn, wd, bm):
    grid = B // bm

    def rep(shape):
        return pl.BlockSpec(shape, lambda i: (0, 0))

    def blk(width):
        return pl.BlockSpec((bm, width), lambda i: (i, 0))

    return pl.pallas_call(
        _tc3_body,
        grid=(grid,),
        in_specs=[
            blk(FD),
            rep((NW, FD)), blk(GRU),
            blk(1), blk(1), blk(1), blk(1), blk(1),
            rep((FD, GRU)), rep((FD, GRU)), rep((FD, GRU)),
            rep((GRU, GRU)), rep((GRU, GRU)), rep((GRU, GRU)),
            rep((2, GRU)), rep((2, GRU)),
            rep((GRU, PW)), rep((2, PW)), rep((1, PW)),
            rep((GRU, PW)), rep((2, PW)), rep((1, PW)),
            rep((GRU, 2)), rep((2, 2)), rep((1, 2)),
        ],
        out_specs=[
            blk(GRU), blk(PW), blk(PW), blk(1), blk(1),
        ],
        out_shape=[
            jax.ShapeDtypeStruct((B, GRU), jnp.float32),
            jax.ShapeDtypeStruct((B, PW), jnp.float32),
            jax.ShapeDtypeStruct((B, PW), jnp.float32),
            jax.ShapeDtypeStruct((B, 1), jnp.float32),
            jax.ShapeDtypeStruct((B, 1), jnp.float32),
        ],
    )(oflat, z, hp, ci, cr, it, rt, nn, *wd)


def _attn_mat(a):
    amat = jnp.zeros((FD, 2 * HEADS), jnp.float32)
    for h in range(HEADS):
        amat = amat.at[H1 * h:H1 * (h + 1), h].set(a[h, :H1])
        amat = amat.at[H1 * h:H1 * (h + 1), h + HEADS].set(a[h, H1:])
    return amat


def kernel(dynamic, cI, cR, N, I, R, edge_index, W1, a1, W2, a2, Wih, Whh,
           bih, bhh, WI, bI, WR, bR, Wsir, bsir):
    f32 = jnp.float32
    src = edge_index[0]
    dst = edge_index[1]
    pad = jnp.zeros((EPAD - E,), jnp.int32)
    srcp = jnp.concatenate([src, pad])
    dstp = jnp.concatenate([dst, pad])

    w1t = W1.reshape(FD, IN_DIM).T
    w2t = W2.reshape(FD, FD).T
    a1m = _attn_mat(a1)
    a2m = _attn_mat(a2)

    wir = Wih[0:GRU].T
    wiz = Wih[GRU:2 * GRU].T
    win = Wih[2 * GRU:].T
    whr = Whh[0:GRU].T
    whz = Whh[GRU:2 * GRU].T
    whn = Whh[2 * GRU:].T
    brz = jnp.stack([bih[0:GRU] + bhh[0:GRU], bih[GRU:2 * GRU] + bhh[GRU:2 * GRU]])
    bn = jnp.stack([bih[2 * GRU:], bhh[2 * GRU:]])
    wd = (wir, wiz, win, whr, whz, whn, brz, bn,
          WI[:, :GRU].T, WI[:, GRU:].T, bI.reshape(1, PW),
          WR[:, :GRU].T, WR[:, GRU:].T, bR.reshape(1, PW),
          Wsir[:, :GRU].T, Wsir[:, GRU:].T, bsir.reshape(1, 2))

    def split_scores(s):
        s1f = s[:, :HEADS].reshape(B * HEADS)
        s2f = jnp.concatenate(
            [s[:, HEADS:], jnp.zeros((BPAD - B, HEADS), f32)]).reshape(-1)
        return s1f, s2f

    h1_all, sc1_all = _tc1(dynamic.reshape(T * B, IN_DIM), w1t, a1m, 2000)
    h1_all = h1_all.reshape(T, B, FD)
    sc1_all = sc1_all.reshape(T, B, 2 * HEADS)

    h = jnp.zeros((B, GRU), f32)
    pIs, pRs, fIs, fRs = [], [], [], []
    for t in range(T):
        s1f, s2f = split_scores(sc1_all[t])
        o1, z1 = _sc_edge(h1_all[t], s1f, s2f, srcp, dstp)
        h2, sc2 = _tc2(o1.reshape(BPAD, FD), z1, w2t, a2m, 2000)
        s1f, s2f = split_scores(sc2)
        o2, z2 = _sc_edge(h2, s1f, s2f, srcp, dstp)
        h, pI, pR, fI, fR = _tc3(o2.reshape(BPAD, FD), z2, h, cI[t], cR[t],
                                 I[t], R[t], N, wd, 2000)
        pIs.append(pI)
        pRs.append(pR)
        fIs.append(fI)
        fRs.append(fR)

    pred_I = jnp.stack(pIs, axis=0)
    pred_R = jnp.stack(pRs, axis=0)
    phy_I = jnp.broadcast_to(jnp.stack(fIs, axis=0), pred_I.shape)
    phy_R = jnp.broadcast_to(jnp.stack(fRs, axis=0), pred_R.shape)
    return (pred_I, pred_R, phy_I, phy_R, h)
